# baseline (speedup 1.0000x reference)
import functools

import jax
import jax.numpy as jnp
from jax import lax
from jax.experimental import pallas as pl
from jax.experimental.pallas import tpu as pltpu
from jax.experimental.pallas import tpu_sc as plsc


def _make_pack_pairs(V, D):
    BN = 2048
    NB = pl.cdiv(V, BN)
    V2 = NB * (BN // 2)

    def body(x_ref, x2_ref, y_ref, y2_ref):
        eye = jnp.eye(D, dtype=jnp.float32)

        def mxu_t(x):
            return jax.lax.dot_general(x, eye, (((0,), (0,)), ((), ())),
                                       preferred_element_type=jnp.float32)

        for xr, yr in ((x_ref, y_ref), (x2_ref, y2_ref)):
            x = xr[...]
            yr[:, 0:D] = mxu_t(x[:, 0:BN // 2])
            yr[:, D:2 * D] = mxu_t(x[:, BN // 2:BN])

    return V2, pl.pallas_call(
        body,
        grid=(NB,),
        in_specs=[pl.BlockSpec((D, BN), lambda i: (0, i)),
                  pl.BlockSpec((D, BN), lambda i: (0, i))],
        out_specs=[pl.BlockSpec((BN // 2, 2 * D), lambda i: (i, 0)),
                   pl.BlockSpec((BN // 2, 2 * D), lambda i: (i, 0))],
        out_shape=[jax.ShapeDtypeStruct((V2, 2 * D), jnp.float32),
                   jax.ShapeDtypeStruct((V2, 2 * D), jnp.float32)],
    )


def _make_sc_kernel(V2, D, B, C):
    info = plsc.get_sparse_core_info()
    NC, NS, L = info.num_cores, info.num_subcores, info.num_lanes
    NW = NC * NS
    assert B % NW == 0
    b_per_w = B // NW
    CH = 128
    assert b_per_w % CH == 0
    n_chunks = b_per_w // CH
    W = 2 * D
    mesh = plsc.VectorSubcoreMesh(core_axis_name="c", subcore_axis_name="s")

    @functools.partial(
        pl.kernel,
        out_type=jax.ShapeDtypeStruct((B * C,), jnp.float32),
        mesh=mesh,
        compiler_params=pltpu.CompilerParams(
            needs_layout_passes=False, use_tc_tiling_on_sc=False),
        scratch_types=[
            pltpu.VMEM((CH,), jnp.int32),
            pltpu.VMEM((CH * C,), jnp.int32),
            pltpu.VMEM((CH,), jnp.int32),
            pltpu.VMEM((CH * C,), jnp.int32),
            pltpu.VMEM((CH, W), jnp.float32),
            pltpu.VMEM((CH * C, W), jnp.float32),
            pltpu.VMEM((CH * C,), jnp.float32),
            pltpu.SemaphoreType.DMA,
            pltpu.SemaphoreType.DMA,
        ],
    )
    def k(tline_hbm, cline_hbm, tpar_hbm, cpar_hbm, ttab_hbm, ctab_hbm, out_hbm,
          lin_t, lin_c, par_t, par_c, rows_t, rows_c, out_v, sem_t, sem_c):
        wid = lax.axis_index("s") * NC + lax.axis_index("c")

        for ck in range(n_chunks):
            base = wid * b_per_w + ck * CH
            pltpu.sync_copy(tline_hbm.at[pl.ds(base, CH)], lin_t)
            pltpu.sync_copy(cline_hbm.at[pl.ds(base * C, CH * C)], lin_c)
            pltpu.sync_copy(tpar_hbm.at[pl.ds(base, CH)], par_t)
            pltpu.sync_copy(cpar_hbm.at[pl.ds(base * C, CH * C)], par_c)
            cp_t = pltpu.async_copy(ttab_hbm.at[lin_t], rows_t, sem_t)
            cp_c = pltpu.async_copy(ctab_hbm.at[lin_c], rows_c, sem_c)
            cp_t.wait()
            cp_c.wait()

            lanes = lax.iota(jnp.int32, L)

            def group_body(g):
                trow = g * L + lanes
                toff = plsc.load_gather(par_t, [trow])
                accs = [jnp.zeros((L,), jnp.float32) for _ in range(C)]
                crows, coffs = [], []
                for c in range(C):
                    crow = trow * C + c
                    crows.append(crow)
                    coffs.append(plsc.load_gather(par_c, [crow]))
                for d in range(D):
                    tv = plsc.load_gather(rows_t, [trow, toff + d])
                    for c in range(C):
                        cv = plsc.load_gather(rows_c, [crows[c], coffs[c] + d])
                        accs[c] = accs[c] + tv * cv
                for c in range(C):
                    plsc.store_scatter(out_v, [trow * C + c], accs[c])

            lax.fori_loop(0, CH // L, lambda g, _: (group_body(g), 0)[1], 0)
            pltpu.sync_copy(out_v, out_hbm.at[pl.ds(base * C, CH * C)])

    return k


def kernel(target, context, target_table, context_table):
    if target.ndim == 2:
        target = jnp.squeeze(target, axis=1)
    V, D = target_table.shape
    B = target.shape[0]
    C = context.shape[1]
    V2, pack = _make_pack_pairs(V, D)
    tpk, cpk = pack(target_table.T, context_table.T)
    tidx = target.astype(jnp.int32)
    cidx = context.reshape(-1).astype(jnp.int32)

    def line_off(v):
        return (v >> 11) * 1024 + (v & 1023), ((v >> 10) & 1) * D

    tl, to = line_off(tidx)
    cl, co = line_off(cidx)
    k = _make_sc_kernel(V2, D, B, C)
    out = k(tl, cl, to, co, tpk, cpk)
    return out.reshape(B, C)


# speedup vs baseline: 1.0773x; 1.0773x over previous
import functools

import jax
import jax.numpy as jnp
from jax import lax
from jax.experimental import pallas as pl
from jax.experimental.pallas import tpu as pltpu
from jax.experimental.pallas import tpu_sc as plsc


def _make_pack(V, D):
    BN = 2048
    NB = pl.cdiv(V, BN)
    V4 = NB * (BN // 4)
    H = D // 2

    def body(x_ref, x2_ref, y_ref, y2_ref):
        for xr, yr in ((x_ref, y_ref), (x2_ref, y2_ref)):
            x = xr[...]
            ulo = lax.bitcast_convert_type(x[0:H, :], jnp.uint32)
            uhi = lax.bitcast_convert_type(x[H:D, :], jnp.uint32)
            w = ((ulo + 0x8000) >> 16) | ((uhi + 0x8000) & jnp.uint32(0xFFFF0000))
            w = lax.bitcast_convert_type(w, jnp.int32)
            wt = w.T
            Q = BN // 4
            yr[...] = jnp.concatenate(
                [wt[Q * q:Q * (q + 1), :] for q in range(4)], axis=1)

    return V4, pl.pallas_call(
        body,
        grid=(NB,),
        in_specs=[pl.BlockSpec((D, BN), lambda i: (0, i)),
                  pl.BlockSpec((D, BN), lambda i: (0, i))],
        out_specs=[pl.BlockSpec((BN // 4, 2 * D), lambda i: (i, 0)),
                   pl.BlockSpec((BN // 4, 2 * D), lambda i: (i, 0))],
        out_shape=[jax.ShapeDtypeStruct((V4, 2 * D), jnp.int32),
                   jax.ShapeDtypeStruct((V4, 2 * D), jnp.int32)],
    )


def _make_sc_kernel(V4, D, B, C):
    info = plsc.get_sparse_core_info()
    NC, NS, L = info.num_cores, info.num_subcores, info.num_lanes
    NW = NC * NS
    assert B % NW == 0
    b_per_w = B // NW
    CH = 128
    assert b_per_w % CH == 0
    n_chunks = b_per_w // CH
    W = 2 * D
    H = D // 2
    HIMASK = jnp.int32(-65536)
    mesh = plsc.VectorSubcoreMesh(core_axis_name="c", subcore_axis_name="s")

    @functools.partial(
        pl.kernel,
        out_type=jax.ShapeDtypeStruct((B * C,), jnp.float32),
        mesh=mesh,
        compiler_params=pltpu.CompilerParams(
            needs_layout_passes=False, use_tc_tiling_on_sc=False),
        scratch_types=[
            pltpu.VMEM((CH,), jnp.int32),
            pltpu.VMEM((CH * C,), jnp.int32),
            pltpu.VMEM((CH,), jnp.int32),
            pltpu.VMEM((CH * C,), jnp.int32),
            pltpu.VMEM((CH, W), jnp.int32),
            pltpu.VMEM((CH * C, W), jnp.int32),
            pltpu.VMEM((CH * C,), jnp.float32),
            pltpu.SemaphoreType.DMA,
            pltpu.SemaphoreType.DMA,
        ],
    )
    def k(tline_hbm, cline_hbm, tcol_hbm, ccol_hbm, ttab_hbm, ctab_hbm, out_hbm,
          lin_t, lin_c, col_t, col_c, rows_t, rows_c, out_v, sem_t, sem_c):
        wid = lax.axis_index("s") * NC + lax.axis_index("c")

        def unpack(wv):
            flo = plsc.bitcast(wv << 16, jnp.float32)
            fhi = plsc.bitcast(wv & HIMASK, jnp.float32)
            return flo, fhi

        for ck in range(n_chunks):
            base = wid * b_per_w + ck * CH
            pltpu.sync_copy(tline_hbm.at[pl.ds(base, CH)], lin_t)
            pltpu.sync_copy(cline_hbm.at[pl.ds(base * C, CH * C)], lin_c)
            pltpu.sync_copy(tcol_hbm.at[pl.ds(base, CH)], col_t)
            pltpu.sync_copy(ccol_hbm.at[pl.ds(base * C, CH * C)], col_c)
            cp_t = pltpu.async_copy(ttab_hbm.at[lin_t], rows_t, sem_t)
            cp_c = pltpu.async_copy(ctab_hbm.at[lin_c], rows_c, sem_c)
            cp_t.wait()
            cp_c.wait()

            lanes = lax.iota(jnp.int32, L)

            def group_body(g):
                trow = g * L + lanes
                toff = plsc.load_gather(col_t, [trow])
                accs = [jnp.zeros((L,), jnp.float32) for _ in range(C)]
                crows, coffs = [], []
                for c in range(C):
                    crow = trow * C + c
                    crows.append(crow)
                    coffs.append(plsc.load_gather(col_c, [crow]))
                for dw in range(H):
                    tlo, thi = unpack(plsc.load_gather(rows_t, [trow, toff + dw]))
                    for c in range(C):
                        clo, chi = unpack(
                            plsc.load_gather(rows_c, [crows[c], coffs[c] + dw]))
                        accs[c] = accs[c] + tlo * clo + thi * chi
                for c in range(C):
                    plsc.store_scatter(out_v, [trow * C + c], accs[c])

            lax.fori_loop(0, CH // L, lambda g, _: (group_body(g), 0)[1], 0)
            pltpu.sync_copy(out_v, out_hbm.at[pl.ds(base * C, CH * C)])

    return k


def kernel(target, context, target_table, context_table):
    if target.ndim == 2:
        target = jnp.squeeze(target, axis=1)
    V, D = target_table.shape
    B = target.shape[0]
    C = context.shape[1]
    V4, pack = _make_pack(V, D)
    tpk, cpk = pack(target_table.T, context_table.T)
    tidx = target.astype(jnp.int32)
    cidx = context.reshape(-1).astype(jnp.int32)

    def line_col(v):
        line = (v >> 11) * 512 + (v & 511)
        col = ((v >> 9) & 3) * (D // 2)
        return line, col

    tl, tc_ = line_col(tidx)
    cl, cc_ = line_col(cidx)
    k = _make_sc_kernel(V4, D, B, C)
    out = k(tl, cl, tc_, cc_, tpk, cpk)
    return out.reshape(B, C)
